# baseline (device time: 78721 ns/iter reference)
import jax
import jax.numpy as jnp
from jax import lax
from jax.experimental import pallas as pl
from jax.experimental.pallas import tpu as pltpu

N_DEV = 4


def kernel(x):
    m_per, n = x.shape
    half = m_per // 2

    def body(x_ref, out_ref, comm_f, comm_b, s_sems, r_sems, cp_sems, own_sem):
        my_pos = lax.axis_index("i")
        left = (my_pos - 1) % N_DEV
        right = (my_pos + 1) % N_DEV

        barrier_sem = pltpu.get_barrier_semaphore()
        for nbr in [left, right]:
            pl.semaphore_signal(
                barrier_sem, inc=1,
                device_id=(nbr,), device_id_type=pl.DeviceIdType.MESH,
            )
        pl.semaphore_wait(barrier_sem, 2)

        def remote(src, dst_sl, target, i):
            return pltpu.make_async_remote_copy(
                src_ref=src,
                dst_ref=out_ref.at[dst_sl],
                send_sem=s_sems.at[i],
                recv_sem=r_sems.at[i],
                device_id=(target,),
                device_id_type=pl.DeviceIdType.MESH,
            )

        bot = pl.ds(my_pos * m_per + half, half)
        top = pl.ds(my_pos * m_per, half)
        f0 = pltpu.make_async_remote_copy(
            src_ref=x_ref.at[pl.ds(0, half)], dst_ref=comm_f,
            send_sem=s_sems.at[0], recv_sem=r_sems.at[0],
            device_id=(right,), device_id_type=pl.DeviceIdType.MESH,
        )
        b0 = pltpu.make_async_remote_copy(
            src_ref=x_ref.at[pl.ds(half, half)], dst_ref=comm_b,
            send_sem=s_sems.at[1], recv_sem=r_sems.at[1],
            device_id=(left,), device_id_type=pl.DeviceIdType.MESH,
        )
        f1 = remote(x_ref.at[pl.ds(half, half)], bot, right, 2)
        b1 = remote(x_ref.at[pl.ds(0, half)], top, left, 3)
        f0.start()
        b0.start()
        f1.start()
        b1.start()

        own = pltpu.make_async_copy(
            x_ref, out_ref.at[pl.ds(my_pos * m_per, m_per)], own_sem
        )
        own.start()

        l_top = pl.ds(left * m_per, half)
        r_bot = pl.ds(right * m_per + half, half)
        f0.wait_recv()
        f2 = remote(comm_f, l_top, right, 4)
        f2.start()
        cp_f = pltpu.make_async_copy(comm_f, out_ref.at[l_top], cp_sems.at[0])
        cp_f.start()
        b0.wait_recv()
        b2 = remote(comm_b, r_bot, left, 5)
        b2.start()
        cp_b = pltpu.make_async_copy(comm_b, out_ref.at[r_bot], cp_sems.at[1])
        cp_b.start()

        f1.wait_recv()
        b1.wait_recv()
        f2.wait_recv()
        b2.wait_recv()

        own.wait()
        cp_f.wait()
        cp_b.wait()
        for r in [f0, b0, f1, b1, f2, b2]:
            r.wait_send()

    x = pltpu.with_memory_space_constraint(x, pltpu.MemorySpace.HBM)
    return pl.pallas_call(
        body,
        out_shape=jax.ShapeDtypeStruct((N_DEV * m_per, n), x.dtype),
        in_specs=[pl.BlockSpec(memory_space=pltpu.MemorySpace.HBM)],
        out_specs=pl.BlockSpec(memory_space=pltpu.MemorySpace.HBM),
        scratch_shapes=[
            pltpu.VMEM((half, n), x.dtype),
            pltpu.VMEM((half, n), x.dtype),
            pltpu.SemaphoreType.DMA((6,)),
            pltpu.SemaphoreType.DMA((6,)),
            pltpu.SemaphoreType.DMA((2,)),
            pltpu.SemaphoreType.DMA,
        ],
        compiler_params=pltpu.CompilerParams(collective_id=0),
    )(x)


# device time: 78677 ns/iter; 1.0006x vs baseline; 1.0006x over previous
import jax
import jax.numpy as jnp
from jax import lax
from jax.experimental import pallas as pl
from jax.experimental.pallas import tpu as pltpu

N_DEV = 4


def kernel(x):
    m_per, n = x.shape
    half = m_per // 2

    def body(x_ref, out_ref, s_sems, r_sems, own_sem):
        my_pos = lax.axis_index("i")
        left = (my_pos - 1) % N_DEV
        right = (my_pos + 1) % N_DEV

        barrier_sem = pltpu.get_barrier_semaphore()
        for nbr in [left, right]:
            pl.semaphore_signal(
                barrier_sem, inc=1,
                device_id=(nbr,), device_id_type=pl.DeviceIdType.MESH,
            )
        pl.semaphore_wait(barrier_sem, 2)

        def remote(src, dst_sl, target, i):
            return pltpu.make_async_remote_copy(
                src_ref=src,
                dst_ref=out_ref.at[dst_sl],
                send_sem=s_sems.at[i],
                recv_sem=r_sems.at[i],
                device_id=(target,),
                device_id_type=pl.DeviceIdType.MESH,
            )

        top = pl.ds(my_pos * m_per, half)
        bot = pl.ds(my_pos * m_per + half, half)
        f0 = remote(x_ref.at[pl.ds(0, half)], top, right, 0)
        b0 = remote(x_ref.at[pl.ds(half, half)], bot, left, 1)
        f1 = remote(x_ref.at[pl.ds(half, half)], bot, right, 2)
        b1 = remote(x_ref.at[pl.ds(0, half)], top, left, 3)
        f0.start()
        b0.start()
        f1.start()
        b1.start()

        own = pltpu.make_async_copy(
            x_ref, out_ref.at[pl.ds(my_pos * m_per, m_per)], own_sem
        )
        own.start()

        l_top = pl.ds(left * m_per, half)
        r_bot = pl.ds(right * m_per + half, half)
        f0.wait_recv()
        f2 = remote(out_ref.at[l_top], l_top, right, 4)
        f2.start()
        b0.wait_recv()
        b2 = remote(out_ref.at[r_bot], r_bot, left, 5)
        b2.start()

        f1.wait_recv()
        b1.wait_recv()
        f2.wait_recv()
        b2.wait_recv()

        own.wait()
        for r in [f0, b0, f1, b1, f2, b2]:
            r.wait_send()

    x = pltpu.with_memory_space_constraint(x, pltpu.MemorySpace.HBM)
    return pl.pallas_call(
        body,
        out_shape=jax.ShapeDtypeStruct((N_DEV * m_per, n), x.dtype),
        in_specs=[pl.BlockSpec(memory_space=pltpu.MemorySpace.HBM)],
        out_specs=pl.BlockSpec(memory_space=pltpu.MemorySpace.HBM),
        scratch_shapes=[
            pltpu.SemaphoreType.DMA((6,)),
            pltpu.SemaphoreType.DMA((6,)),
            pltpu.SemaphoreType.DMA,
        ],
        compiler_params=pltpu.CompilerParams(collective_id=0),
    )(x)
